# initial kernel scaffold (unmeasured)
import jax
import jax.numpy as jnp
from jax import lax
from jax.experimental import pallas as pl
from jax.experimental.pallas import tpu as pltpu

N_DEV = 16
M = 1024
N = 1024
CHUNK = M // N_DEV


def kernel(x, w_mat):
    def body(x_ref, w_ref, out_ref, gather_ref, send_sems, recv_sems_p1,
             recv_sems_p2):
        my = lax.axis_index("i")

        out_ref[...] = jnp.dot(
            x_ref[...], w_ref[...], preferred_element_type=jnp.float32
        )

        gather_ref[pl.ds(my, 1)] = out_ref[pl.ds(my * CHUNK, CHUNK), :].reshape(
            1, CHUNK, N
        )

        p1 = []
        for o in range(1, N_DEV):
            peer = lax.rem(my + o, N_DEV)
            rdma = pltpu.make_async_remote_copy(
                src_ref=out_ref.at[pl.ds(peer * CHUNK, CHUNK), :],
                dst_ref=gather_ref.at[my],
                send_sem=send_sems.at[o - 1],
                recv_sem=recv_sems_p1.at[o - 1],
                device_id=(peer,),
                device_id_type=pl.DeviceIdType.MESH,
            )
            rdma.start()
            p1.append(rdma)
        for rdma in p1:
            rdma.wait_recv()

        reduced = jnp.sum(gather_ref[...], axis=0)

        for rdma in p1:
            rdma.wait_send()

        out_ref[pl.ds(my * CHUNK, CHUNK), :] = reduced

        p2 = []
        for o in range(1, N_DEV):
            peer = lax.rem(my + o, N_DEV)
            rdma = pltpu.make_async_remote_copy(
                src_ref=out_ref.at[pl.ds(my * CHUNK, CHUNK), :],
                dst_ref=out_ref.at[pl.ds(my * CHUNK, CHUNK), :],
                send_sem=send_sems.at[o - 1],
                recv_sem=recv_sems_p2.at[o - 1],
                device_id=(peer,),
                device_id_type=pl.DeviceIdType.MESH,
            )
            rdma.start()
            p2.append(rdma)
        for rdma in p2:
            rdma.wait_recv()
        for rdma in p2:
            rdma.wait_send()

        out_ref[...] = jnp.maximum(out_ref[...], 0.0)

    return pl.pallas_call(
        body,
        out_shape=jax.ShapeDtypeStruct((M, N), jnp.float32),
        in_specs=[
            pl.BlockSpec(memory_space=pltpu.VMEM),
            pl.BlockSpec(memory_space=pltpu.VMEM),
        ],
        out_specs=pl.BlockSpec(memory_space=pltpu.VMEM),
        scratch_shapes=[
            pltpu.VMEM((N_DEV, CHUNK, N), jnp.float32),
            pltpu.SemaphoreType.DMA((N_DEV - 1,)),
            pltpu.SemaphoreType.DMA((N_DEV - 1,)),
            pltpu.SemaphoreType.DMA((N_DEV - 1,)),
        ],
        compiler_params=pltpu.CompilerParams(collective_id=0),
    )(x, w_mat)


# baseline (device time: 116109 ns/iter reference)
import jax
import jax.numpy as jnp
from jax import lax
from jax.experimental import pallas as pl
from jax.experimental.pallas import tpu as pltpu

N_DEV = 16
M = 1024
N = 1024
CHUNK = M // N_DEV


def kernel(x, w_mat):
    def body(x_ref, w_ref, out_ref, gather_ref, send_sems, recv_sems_p1,
             recv_sems_p2):
        my = lax.axis_index("i")

        out_ref[...] = jnp.dot(
            x_ref[...], w_ref[...], preferred_element_type=jnp.float32
        )

        gather_ref[pl.ds(my, 1)] = out_ref[pl.ds(my * CHUNK, CHUNK), :].reshape(
            1, CHUNK, N
        )

        p1 = []
        for o in range(1, N_DEV):
            peer = lax.rem(my + o, N_DEV)
            rdma = pltpu.make_async_remote_copy(
                src_ref=out_ref.at[pl.ds(peer * CHUNK, CHUNK), :],
                dst_ref=gather_ref.at[my],
                send_sem=send_sems.at[o - 1],
                recv_sem=recv_sems_p1.at[o - 1],
                device_id=(peer,),
                device_id_type=pl.DeviceIdType.MESH,
            )
            rdma.start()
            p1.append(rdma)
        for rdma in p1:
            rdma.wait_recv()

        reduced = jnp.sum(gather_ref[...], axis=0)

        for rdma in p1:
            rdma.wait_send()

        out_ref[pl.ds(my * CHUNK, CHUNK), :] = reduced

        p2 = []
        for o in range(1, N_DEV):
            peer = lax.rem(my + o, N_DEV)
            rdma = pltpu.make_async_remote_copy(
                src_ref=out_ref.at[pl.ds(my * CHUNK, CHUNK), :],
                dst_ref=out_ref.at[pl.ds(my * CHUNK, CHUNK), :],
                send_sem=send_sems.at[o - 1],
                recv_sem=recv_sems_p2.at[o - 1],
                device_id=(peer,),
                device_id_type=pl.DeviceIdType.MESH,
            )
            rdma.start()
            p2.append(rdma)
        for rdma in p2:
            rdma.wait_recv()
        for rdma in p2:
            rdma.wait_send()

        out_ref[...] = jnp.maximum(out_ref[...], 0.0)

    return pl.pallas_call(
        body,
        out_shape=jax.ShapeDtypeStruct((M, N), jnp.float32),
        in_specs=[
            pl.BlockSpec(memory_space=pltpu.VMEM),
            pl.BlockSpec(memory_space=pltpu.VMEM),
        ],
        out_specs=pl.BlockSpec(memory_space=pltpu.VMEM),
        scratch_shapes=[
            pltpu.VMEM((N_DEV, CHUNK, N), jnp.float32),
            pltpu.SemaphoreType.DMA((N_DEV - 1,)),
            pltpu.SemaphoreType.DMA((N_DEV - 1,)),
            pltpu.SemaphoreType.DMA((N_DEV - 1,)),
        ],
    )(x, w_mat)


# device time: 85568 ns/iter; 1.3569x vs baseline; 1.3569x over previous
import jax
import jax.numpy as jnp
from jax import lax
from jax.experimental import pallas as pl
from jax.experimental.pallas import tpu as pltpu

N_DEV = 16
M = 1024
N = 1024
QROWS = 256
BROWS = 64


def kernel(x, w_mat):
    def body(x_ref, w_ref, out_ref, pg_ref, zg_ref,
             s1s, s1r, s2s, s2r, s3s, s3r, s4s, s4r):
        my = lax.axis_index("i")
        z = my // 4
        pp = lax.rem(my, 4)

        out_ref[...] = jnp.dot(
            x_ref[...], w_ref[...], preferred_element_type=jnp.float32
        )

        pg_ref[pl.ds(pp, 1)] = out_ref[pl.ds(pp * QROWS, QROWS), :].reshape(
            1, QROWS, N
        )
        st1 = []
        for o in range(1, 4):
            j = lax.rem(pp + o, 4)
            peer = 4 * z + j
            rdma = pltpu.make_async_remote_copy(
                src_ref=out_ref.at[pl.ds(j * QROWS, QROWS), :],
                dst_ref=pg_ref.at[pp],
                send_sem=s1s.at[o - 1],
                recv_sem=s1r.at[o - 1],
                device_id=(peer,),
                device_id_type=pl.DeviceIdType.MESH,
            )
            rdma.start()
            st1.append(rdma)
        for rdma in st1:
            rdma.wait_recv()
        out_ref[pl.ds(pp * QROWS, QROWS), :] = jnp.sum(pg_ref[...], axis=0)

        zg_ref[pl.ds(z, 1)] = out_ref[
            pl.ds(pp * QROWS + z * BROWS, BROWS), :
        ].reshape(1, BROWS, N)
        st2 = []
        for o in range(1, 4):
            k = lax.rem(z + o, 4)
            peer = 4 * k + pp
            rdma = pltpu.make_async_remote_copy(
                src_ref=out_ref.at[pl.ds(pp * QROWS + k * BROWS, BROWS), :],
                dst_ref=zg_ref.at[z],
                send_sem=s2s.at[o - 1],
                recv_sem=s2r.at[o - 1],
                device_id=(peer,),
                device_id_type=pl.DeviceIdType.MESH,
            )
            rdma.start()
            st2.append(rdma)
        for rdma in st2:
            rdma.wait_recv()
        out_ref[pl.ds(pp * QROWS + z * BROWS, BROWS), :] = jnp.sum(
            zg_ref[...], axis=0
        )

        st3 = []
        for o in range(1, 4):
            k = lax.rem(z + o, 4)
            peer = 4 * k + pp
            rdma = pltpu.make_async_remote_copy(
                src_ref=out_ref.at[pl.ds(pp * QROWS + z * BROWS, BROWS), :],
                dst_ref=out_ref.at[pl.ds(pp * QROWS + z * BROWS, BROWS), :],
                send_sem=s3s.at[o - 1],
                recv_sem=s3r.at[o - 1],
                device_id=(peer,),
                device_id_type=pl.DeviceIdType.MESH,
            )
            rdma.start()
            st3.append(rdma)
        for rdma in st3:
            rdma.wait_recv()

        st4 = []
        for o in range(1, 4):
            j = lax.rem(pp + o, 4)
            peer = 4 * z + j
            rdma = pltpu.make_async_remote_copy(
                src_ref=out_ref.at[pl.ds(pp * QROWS, QROWS), :],
                dst_ref=out_ref.at[pl.ds(pp * QROWS, QROWS), :],
                send_sem=s4s.at[o - 1],
                recv_sem=s4r.at[o - 1],
                device_id=(peer,),
                device_id_type=pl.DeviceIdType.MESH,
            )
            rdma.start()
            st4.append(rdma)
        for rdma in st4:
            rdma.wait_recv()

        for rdma in st1 + st2 + st3 + st4:
            rdma.wait_send()

        out_ref[...] = jnp.maximum(out_ref[...], 0.0)

    sem3 = pltpu.SemaphoreType.DMA((3,))
    return pl.pallas_call(
        body,
        out_shape=jax.ShapeDtypeStruct((M, N), jnp.float32),
        in_specs=[
            pl.BlockSpec(memory_space=pltpu.VMEM),
            pl.BlockSpec(memory_space=pltpu.VMEM),
        ],
        out_specs=pl.BlockSpec(memory_space=pltpu.VMEM),
        scratch_shapes=[
            pltpu.VMEM((4, QROWS, N), jnp.float32),
            pltpu.VMEM((4, BROWS, N), jnp.float32),
            sem3, sem3,
            sem3, sem3,
            sem3, sem3,
            sem3, sem3,
        ],
    )(x, w_mat)


# device time: 62912 ns/iter; 1.8456x vs baseline; 1.3601x over previous
import jax
import jax.numpy as jnp
from jax import lax
from jax.experimental import pallas as pl
from jax.experimental.pallas import tpu as pltpu

N_DEV = 16
M = 1024
N = 1024
C0 = 640
C1 = N - C0
QROWS = 256
BROWS = 64


def kernel(x, w_mat):
    def body(x_ref, w_ref, out_ref, pg0, zg0, zg1, pg1, *sems):
        (h0s1s, h0s1r, h0s2s, h0s2r, h0s3s, h0s3r, h0s4s, h0s4r,
         h1s1s, h1s1r, h1s2s, h1s2r, h1s3s, h1s3r, h1s4s, h1s4r) = sems
        my = lax.axis_index("i")
        z = my // 4
        pp = lax.rem(my, 4)

        out_ref[...] = jnp.dot(
            x_ref[...], w_ref[...], preferred_element_type=jnp.float32
        )

        def send_group(src_row, rows, col, width, dst_ref, ssem, rsem, peers):
            group = []
            for o, peer in enumerate(peers):
                rdma = pltpu.make_async_remote_copy(
                    src_ref=out_ref.at[pl.ds(src_row(o), rows),
                                       pl.ds(col, width)],
                    dst_ref=dst_ref,
                    send_sem=ssem.at[o],
                    recv_sem=rsem.at[o],
                    device_id=(peer,),
                    device_id_type=pl.DeviceIdType.MESH,
                )
                rdma.start()
                group.append(rdma)
            return group

        plane_peers = [4 * z + lax.rem(pp + o, 4) for o in range(1, 4)]
        z_peers = [4 * lax.rem(z + o, 4) + pp for o in range(1, 4)]
        pj = [lax.rem(pp + o, 4) for o in range(1, 4)]
        zk = [lax.rem(z + o, 4) for o in range(1, 4)]

        pg0[pl.ds(pp, 1)] = out_ref[pl.ds(pp * QROWS, QROWS),
                                    pl.ds(0, C0)].reshape(1, QROWS, C0)
        h0s1 = send_group(lambda o: pj[o] * QROWS, QROWS, 0, C0,
                          pg0.at[pp], h0s1s, h0s1r, plane_peers)
        zg1[pl.ds(z, 1)] = out_ref[pl.ds(z * QROWS, QROWS),
                                   pl.ds(C0, C1)].reshape(1, QROWS, C1)
        h1s1 = send_group(lambda o: zk[o] * QROWS, QROWS, C0, C1,
                          zg1.at[z], h1s1s, h1s1r, z_peers)

        for rdma in h0s1:
            rdma.wait_recv()
        out_ref[pl.ds(pp * QROWS, QROWS), pl.ds(0, C0)] = jnp.sum(
            pg0[...], axis=0
        )
        zg0[pl.ds(z, 1)] = out_ref[pl.ds(pp * QROWS + z * BROWS, BROWS),
                                   pl.ds(0, C0)].reshape(1, BROWS, C0)
        h0s2 = send_group(lambda o: pp * QROWS + zk[o] * BROWS, BROWS, 0, C0,
                          zg0.at[z], h0s2s, h0s2r, z_peers)

        for rdma in h1s1:
            rdma.wait_recv()
        out_ref[pl.ds(z * QROWS, QROWS), pl.ds(C0, C1)] = jnp.sum(
            zg1[...], axis=0
        )
        pg1[pl.ds(pp, 1)] = out_ref[pl.ds(z * QROWS + pp * BROWS, BROWS),
                                    pl.ds(C0, C1)].reshape(1, BROWS, C1)
        h1s2 = send_group(lambda o: z * QROWS + pj[o] * BROWS, BROWS, C0, C1,
                          pg1.at[pp], h1s2s, h1s2r, plane_peers)

        for rdma in h0s2:
            rdma.wait_recv()
        out_ref[pl.ds(pp * QROWS + z * BROWS, BROWS), pl.ds(0, C0)] = jnp.sum(
            zg0[...], axis=0
        )
        h0s3 = send_group(lambda o: pp * QROWS + z * BROWS, BROWS, 0, C0,
                          out_ref.at[pl.ds(pp * QROWS + z * BROWS, BROWS),
                                     pl.ds(0, C0)],
                          h0s3s, h0s3r, z_peers)

        for rdma in h1s2:
            rdma.wait_recv()
        out_ref[pl.ds(z * QROWS + pp * BROWS, BROWS), pl.ds(C0, C1)] = jnp.sum(
            pg1[...], axis=0
        )
        h1s3 = send_group(lambda o: z * QROWS + pp * BROWS, BROWS, C0, C1,
                          out_ref.at[pl.ds(z * QROWS + pp * BROWS, BROWS),
                                     pl.ds(C0, C1)],
                          h1s3s, h1s3r, plane_peers)

        for rdma in h0s3:
            rdma.wait_recv()
        h0s4 = send_group(lambda o: pp * QROWS, QROWS, 0, C0,
                          out_ref.at[pl.ds(pp * QROWS, QROWS), pl.ds(0, C0)],
                          h0s4s, h0s4r, plane_peers)

        for rdma in h1s3:
            rdma.wait_recv()
        h1s4 = send_group(lambda o: z * QROWS, QROWS, C0, C1,
                          out_ref.at[pl.ds(z * QROWS, QROWS), pl.ds(C0, C1)],
                          h1s4s, h1s4r, z_peers)

        for rdma in h0s4:
            rdma.wait_recv()
        for rdma in h1s4:
            rdma.wait_recv()
        for rdma in h0s1 + h0s2 + h0s3 + h0s4 + h1s1 + h1s2 + h1s3 + h1s4:
            rdma.wait_send()

        out_ref[...] = jnp.maximum(out_ref[...], 0.0)

    sem3 = pltpu.SemaphoreType.DMA((3,))
    return pl.pallas_call(
        body,
        out_shape=jax.ShapeDtypeStruct((M, N), jnp.float32),
        in_specs=[
            pl.BlockSpec(memory_space=pltpu.VMEM),
            pl.BlockSpec(memory_space=pltpu.VMEM),
        ],
        out_specs=pl.BlockSpec(memory_space=pltpu.VMEM),
        scratch_shapes=[
            pltpu.VMEM((4, QROWS, C0), jnp.float32),
            pltpu.VMEM((4, BROWS, C0), jnp.float32),
            pltpu.VMEM((4, QROWS, C1), jnp.float32),
            pltpu.VMEM((4, BROWS, C1), jnp.float32),
        ] + [sem3] * 16,
    )(x, w_mat)
